# baseline (speedup 1.0000x reference)
import jax
import jax.numpy as jnp
from jax import lax
from jax.experimental import pallas as pl
from jax.experimental.pallas import tpu as pltpu
from jax.experimental.pallas import tpu_sc as plsc

VOCAB = 1000000
HIDDEN = 32
NUM_FIELD = 26
BATCH = 16384

N_TOT = BATCH * NUM_FIELD
NW = 32
PER_W = N_TOT // NW
CHUNK = 104
ROWS_PER_FIELD = CHUNK // NUM_FIELD
NPAIR = PER_W // (2 * CHUNK)

_LANES = 16
_REC = 2 * HIDDEN
_TROW = 2 * _REC


def _sc_body(x_hbm, tbl_hbm, cb_hbm, out_hbm,
             xv, gidx, cbv, wm, obuf, gsemA, gsemB, osemA, osemB):
  wid = lax.axis_index("c") * 16 + lax.axis_index("s")
  base = wid * PER_W

  pltpu.sync_copy(x_hbm.at[pl.ds(base, PER_W)], xv)
  pltpu.sync_copy(cb_hbm, cbv)

  halfv = jnp.full((_LANES,), VOCAB // 2, jnp.int32)
  def gidx_body(t, _):
    v = xv[pl.ds(t * _LANES, _LANES)]
    gidx[pl.ds(t * _LANES, _LANES)] = jnp.where(v >= halfv, v - halfv, v)
    return 0
  lax.fori_loop(0, PER_W // _LANES, gidx_body, 0)

  fzero = jnp.zeros((_LANES,), jnp.float32)
  klo = lax.iota(jnp.int32, _LANES)
  gsems = (gsemA, gsemB)
  osems = (osemA, osemB)

  def start_gather(c, p):
    pltpu.async_copy(
        tbl_hbm.at[gidx.at[pl.ds(c * CHUNK, CHUNK)]], wm.at[p], gsems[p])

  def wait_gather(p):
    pltpu.make_async_copy(tbl_hbm.at[pl.ds(0, CHUNK)], wm.at[p],
                          gsems[p]).wait()

  def start_write(c, p):
    pltpu.async_copy(
        obuf.at[p], out_hbm.at[pl.ds((base + c * CHUNK) * HIDDEN,
                                     CHUNK * HIDDEN)], osems[p])

  def wait_write(p):
    pltpu.make_async_copy(obuf.at[p],
                          out_hbm.at[pl.ds(0, CHUNK * HIDDEN)],
                          osems[p]).wait()

  def compute(c, p):
    for j in range(NUM_FIELD):
      cb0 = cbv[pl.ds(j * HIDDEN, _LANES)]
      cb1 = cbv[pl.ds(j * HIDDEN + _LANES, _LANES)]

      def row_body(i, _, j=j, cb0=cb0, cb1=cb1):
        r = j + NUM_FIELD * i
        rloc = jnp.full((_LANES,), r, jnp.int32)
        rglob = jnp.full((_LANES,), c * CHUNK + r, jnp.int32)
        v = plsc.load_gather(xv, [rglob])
        wi0 = jnp.where(v >= halfv, jnp.full((_LANES,), 64, jnp.int32),
                        jnp.zeros((_LANES,), jnp.int32)) + klo
        w0 = plsc.load_gather(wm.at[p], [rloc, wi0])
        w1 = plsc.load_gather(wm.at[p], [rloc, wi0 + 16])
        m0 = plsc.load_gather(wm.at[p], [rloc, wi0 + 32])
        m1 = plsc.load_gather(wm.at[p], [rloc, wi0 + 48])
        obuf[p, pl.ds(r * HIDDEN, _LANES)] = jnp.where(m0 != fzero, cb0, w0)
        obuf[p, pl.ds(r * HIDDEN + _LANES, _LANES)] = jnp.where(
            m1 != fzero, cb1, w1)
        return 0

      lax.fori_loop(0, ROWS_PER_FIELD, row_body, 0)

  def pair_body(k, _):
    cA = 2 * k
    cB = 2 * k + 1

    @pl.when(k > 0)
    def _():
      wait_write(0)
    start_gather(cB, 1)
    wait_gather(0)
    compute(cA, 0)
    start_write(cA, 0)

    @pl.when(k > 0)
    def _():
      wait_write(1)

    @pl.when(k < NPAIR - 1)
    def _():
      start_gather(cA + 2, 0)
    wait_gather(1)
    compute(cB, 1)
    start_write(cB, 1)
    return 0

  start_gather(0, 0)
  lax.fori_loop(0, NPAIR, pair_body, 0)
  wait_write(0)
  wait_write(1)


@jax.jit
def kernel(x, codebook_mask, weight, codebook):
  x_flat = x.reshape(N_TOT).astype(jnp.int32)
  h = VOCAB // 2
  mask_f = codebook_mask.astype(jnp.float32)
  tbl = jnp.concatenate(
      [weight[:h], mask_f[:h], weight[h:], mask_f[h:]], axis=1)
  cb_flat = codebook.reshape(NUM_FIELD * HIDDEN)

  mesh = plsc.VectorSubcoreMesh(core_axis_name="c", subcore_axis_name="s")
  out = pl.kernel(
      _sc_body,
      out_type=jax.ShapeDtypeStruct((N_TOT * HIDDEN,), jnp.float32),
      mesh=mesh,
      compiler_params=pltpu.CompilerParams(
          use_tc_tiling_on_sc=True, needs_layout_passes=False),
      scratch_types=[
          pltpu.VMEM((PER_W,), jnp.int32),
          pltpu.VMEM((PER_W,), jnp.int32),
          pltpu.VMEM((NUM_FIELD * HIDDEN,), jnp.float32),
          pltpu.VMEM((2, CHUNK, _TROW), jnp.float32),
          pltpu.VMEM((2, CHUNK * HIDDEN), jnp.float32),
          pltpu.SemaphoreType.DMA,
          pltpu.SemaphoreType.DMA,
          pltpu.SemaphoreType.DMA,
          pltpu.SemaphoreType.DMA,
      ],
  )(x_flat, tbl, cb_flat)
  return out.reshape(BATCH, NUM_FIELD, HIDDEN)


# speedup vs baseline: 1.2395x; 1.2395x over previous
import jax
import jax.numpy as jnp
from jax import lax
from jax.experimental import pallas as pl
from jax.experimental.pallas import tpu as pltpu
from jax.experimental.pallas import tpu_sc as plsc

VOCAB = 1000000
HIDDEN = 32
NUM_FIELD = 26
BATCH = 16384

N_TOT = BATCH * NUM_FIELD
NW = 32
PER_W = N_TOT // NW
CHUNK = 104
ROWS_PER_FIELD = CHUNK // NUM_FIELD
NPAIR = PER_W // (2 * CHUNK)

_LANES = 16
_TROW = 128


def _sc_body(x_hbm, w_hbm, m_hbm, cb_hbm, out_hbm,
             xv, gidx, cbv, wbuf, mbuf, obuf, gsemA, gsemB, osemA, osemB):
  wid = lax.axis_index("c") * 16 + lax.axis_index("s")
  base = wid * PER_W

  pltpu.sync_copy(x_hbm.at[pl.ds(base, PER_W)], xv)
  pltpu.sync_copy(cb_hbm, cbv)

  def gidx_body(t, _):
    gidx[pl.ds(t * _LANES, _LANES)] = xv[pl.ds(t * _LANES, _LANES)] >> 2
    return 0
  lax.fori_loop(0, PER_W // _LANES, gidx_body, 0)

  fzero = jnp.zeros((_LANES,), jnp.float32)
  klo = lax.iota(jnp.int32, _LANES)
  gsems = (gsemA, gsemB)
  osems = (osemA, osemB)

  def start_gather(c, p):
    idx = gidx.at[pl.ds(c * CHUNK, CHUNK)]
    pltpu.async_copy(w_hbm.at[idx], wbuf.at[p], gsems[p])
    pltpu.async_copy(m_hbm.at[idx], mbuf.at[p], gsems[p])

  def wait_gather(p):
    pltpu.make_async_copy(w_hbm.at[pl.ds(0, CHUNK)], wbuf.at[p],
                          gsems[p]).wait()
    pltpu.make_async_copy(m_hbm.at[pl.ds(0, CHUNK)], mbuf.at[p],
                          gsems[p]).wait()

  def start_write(c, p):
    pltpu.async_copy(
        obuf.at[p], out_hbm.at[pl.ds((base + c * CHUNK) * HIDDEN,
                                     CHUNK * HIDDEN)], osems[p])

  def wait_write(p):
    pltpu.make_async_copy(obuf.at[p],
                          out_hbm.at[pl.ds(0, CHUNK * HIDDEN)],
                          osems[p]).wait()

  def compute(c, p):
    for j in range(NUM_FIELD):
      cb0 = cbv[pl.ds(j * HIDDEN, _LANES)]
      cb1 = cbv[pl.ds(j * HIDDEN + _LANES, _LANES)]

      def row_body(i, _, j=j, cb0=cb0, cb1=cb1):
        r = j + NUM_FIELD * i
        rloc = jnp.full((_LANES,), r, jnp.int32)
        rglob = jnp.full((_LANES,), c * CHUNK + r, jnp.int32)
        v = plsc.load_gather(xv, [rglob])
        wi0 = ((v & 3) << 5) + klo
        w0 = plsc.load_gather(wbuf.at[p], [rloc, wi0])
        w1 = plsc.load_gather(wbuf.at[p], [rloc, wi0 + 16])
        m0 = plsc.load_gather(mbuf.at[p], [rloc, wi0])
        m1 = plsc.load_gather(mbuf.at[p], [rloc, wi0 + 16])
        obuf[p, pl.ds(r * HIDDEN, _LANES)] = jnp.where(m0 != fzero, cb0, w0)
        obuf[p, pl.ds(r * HIDDEN + _LANES, _LANES)] = jnp.where(
            m1 != fzero, cb1, w1)
        return 0

      lax.fori_loop(0, ROWS_PER_FIELD, row_body, 0)

  def pair_body(k, _):
    cA = 2 * k
    cB = 2 * k + 1

    @pl.when(k > 0)
    def _():
      wait_write(0)
    start_gather(cB, 1)
    wait_gather(0)
    compute(cA, 0)
    start_write(cA, 0)

    @pl.when(k > 0)
    def _():
      wait_write(1)

    @pl.when(k < NPAIR - 1)
    def _():
      start_gather(cA + 2, 0)
    wait_gather(1)
    compute(cB, 1)
    start_write(cB, 1)
    return 0

  start_gather(0, 0)
  lax.fori_loop(0, NPAIR, pair_body, 0)
  wait_write(0)
  wait_write(1)


@jax.jit
def kernel(x, codebook_mask, weight, codebook):
  x_flat = x.reshape(N_TOT).astype(jnp.int32)
  wtbl = weight.reshape(VOCAB // 4, _TROW)
  mtbl = codebook_mask.astype(jnp.float32).reshape(VOCAB // 4, _TROW)
  cb_flat = codebook.reshape(NUM_FIELD * HIDDEN)

  mesh = plsc.VectorSubcoreMesh(core_axis_name="c", subcore_axis_name="s")
  out = pl.kernel(
      _sc_body,
      out_type=jax.ShapeDtypeStruct((N_TOT * HIDDEN,), jnp.float32),
      mesh=mesh,
      compiler_params=pltpu.CompilerParams(
          use_tc_tiling_on_sc=True, needs_layout_passes=False),
      scratch_types=[
          pltpu.VMEM((PER_W,), jnp.int32),
          pltpu.VMEM((PER_W,), jnp.int32),
          pltpu.VMEM((NUM_FIELD * HIDDEN,), jnp.float32),
          pltpu.VMEM((2, CHUNK, _TROW), jnp.float32),
          pltpu.VMEM((2, CHUNK, _TROW), jnp.float32),
          pltpu.VMEM((2, CHUNK * HIDDEN), jnp.float32),
          pltpu.SemaphoreType.DMA,
          pltpu.SemaphoreType.DMA,
          pltpu.SemaphoreType.DMA,
          pltpu.SemaphoreType.DMA,
      ],
  )(x_flat, wtbl, mtbl, cb_flat)
  return out.reshape(BATCH, NUM_FIELD, HIDDEN)


# speedup vs baseline: 1.5110x; 1.2190x over previous
import jax
import jax.numpy as jnp
from jax import lax
from jax.experimental import pallas as pl
from jax.experimental.pallas import tpu as pltpu
from jax.experimental.pallas import tpu_sc as plsc

VOCAB = 1000000
HIDDEN = 32
NUM_FIELD = 26
BATCH = 16384

N_TOT = BATCH * NUM_FIELD
NW = 32
PER_W = N_TOT // NW
CHUNK = 416
ROWS_PER_FIELD = CHUNK // NUM_FIELD
NPAIR = PER_W // (2 * CHUNK)
SUB = 104
NSUB = CHUNK // SUB

_LANES = 16


def _sc_body(x_hbm, mask_hbm, w_hbm, cb_hbm, out_hbm,
             xv, cbv, wbuf, mbuf, obuf, gsemA, gsemB, osemA, osemB):
  wid = lax.axis_index("c") * 16 + lax.axis_index("s")
  base = wid * PER_W

  pltpu.sync_copy(x_hbm.at[pl.ds(base, PER_W)], xv)
  pltpu.sync_copy(cb_hbm, cbv)

  fzero = jnp.zeros((_LANES,), jnp.float32)
  gsems = (gsemA, gsemB)
  osems = (osemA, osemB)

  def start_gather(c, p):
    for s in range(NSUB):
      idx = xv.at[pl.ds(c * CHUNK + s * SUB, SUB)]
      pltpu.async_copy(w_hbm.at[idx], wbuf.at[p, pl.ds(s * SUB, SUB)],
                       gsems[p])
      pltpu.async_copy(mask_hbm.at[idx], mbuf.at[p, pl.ds(s * SUB, SUB)],
                       gsems[p])

  def wait_gather(p):
    pltpu.make_async_copy(w_hbm.at[pl.ds(0, CHUNK)], wbuf.at[p],
                          gsems[p]).wait()
    pltpu.make_async_copy(mask_hbm.at[pl.ds(0, CHUNK)], mbuf.at[p],
                          gsems[p]).wait()

  def start_write(c, p):
    pltpu.async_copy(obuf.at[p],
                     out_hbm.at[pl.ds(base + c * CHUNK, CHUNK)], osems[p])

  def wait_write(p):
    pltpu.make_async_copy(obuf.at[p], out_hbm.at[pl.ds(0, CHUNK)],
                          osems[p]).wait()

  def compute(p):
    for j in range(NUM_FIELD):
      cb0 = cbv[j, pl.ds(0, _LANES)]
      cb1 = cbv[j, pl.ds(_LANES, _LANES)]

      def row_body(i, _, cb0=cb0, cb1=cb1, j=j):
        r = j + NUM_FIELD * i
        s0 = mbuf[p, r, pl.ds(0, _LANES)] != fzero
        s1 = mbuf[p, r, pl.ds(_LANES, _LANES)] != fzero
        w0 = wbuf[p, r, pl.ds(0, _LANES)]
        w1 = wbuf[p, r, pl.ds(_LANES, _LANES)]
        obuf[p, r, pl.ds(0, _LANES)] = jnp.where(s0, cb0, w0)
        obuf[p, r, pl.ds(_LANES, _LANES)] = jnp.where(s1, cb1, w1)
        return 0

      lax.fori_loop(0, ROWS_PER_FIELD, row_body, 0)

  def pair_body(k, _):
    cA = 2 * k
    cB = 2 * k + 1

    @pl.when(k > 0)
    def _():
      wait_write(0)
    start_gather(cB, 1)
    wait_gather(0)
    compute(0)
    start_write(cA, 0)

    @pl.when(k > 0)
    def _():
      wait_write(1)

    @pl.when(k < NPAIR - 1)
    def _():
      start_gather(cA + 2, 0)
    wait_gather(1)
    compute(1)
    start_write(cB, 1)
    return 0

  start_gather(0, 0)
  lax.fori_loop(0, NPAIR, pair_body, 0)
  wait_write(0)
  wait_write(1)


@jax.jit
def kernel(x, codebook_mask, weight, codebook):
  x_flat = x.reshape(N_TOT).astype(jnp.int32)
  mask_f = codebook_mask.astype(jnp.float32)

  mesh = plsc.VectorSubcoreMesh(core_axis_name="c", subcore_axis_name="s")
  out = pl.kernel(
      _sc_body,
      out_type=jax.ShapeDtypeStruct((N_TOT, HIDDEN), jnp.float32),
      mesh=mesh,
      compiler_params=pltpu.CompilerParams(
          use_tc_tiling_on_sc=False, needs_layout_passes=False),
      scratch_types=[
          pltpu.VMEM((PER_W,), jnp.int32),
          pltpu.VMEM((NUM_FIELD, HIDDEN), jnp.float32),
          pltpu.VMEM((2, CHUNK, HIDDEN), jnp.float32),
          pltpu.VMEM((2, CHUNK, HIDDEN), jnp.float32),
          pltpu.VMEM((2, CHUNK, HIDDEN), jnp.float32),
          pltpu.SemaphoreType.DMA,
          pltpu.SemaphoreType.DMA,
          pltpu.SemaphoreType.DMA,
          pltpu.SemaphoreType.DMA,
      ],
  )(x_flat, mask_f, weight, codebook)
  return out.reshape(BATCH, NUM_FIELD, HIDDEN)
